# initial kernel scaffold (unmeasured)
import jax
import jax.numpy as jnp
from jax import lax
from jax.experimental import pallas as pl
from jax.experimental.pallas import tpu as pltpu

N_DEV = 32
BLK = 128
N_COLS = 8192
N_BLK = 1024


def _a2a_kernel(x_shard):
    k_dim, k_per = x_shard.shape

    def body(x_ref, out_ref, send_sems, recv_sems):
        me = lax.axis_index("i")

        for t in range(N_DEV):
            @pl.when(t != me)
            def _():
                rdma = pltpu.make_async_remote_copy(
                    src_ref=x_ref.at[pl.ds(t * BLK, BLK), :],
                    dst_ref=out_ref.at[:, pl.ds(me * BLK, BLK)],
                    send_sem=send_sems.at[t],
                    recv_sem=recv_sems.at[me],
                    device_id=(t,),
                    device_id_type=pl.DeviceIdType.MESH,
                )
                rdma.start()

        out_ref[:, pl.ds(me * BLK, BLK)] = x_ref[pl.ds(me * BLK, BLK), :]

        for s in range(N_DEV):
            @pl.when(s != me)
            def _():
                recv = pltpu.make_async_remote_copy(
                    src_ref=x_ref.at[pl.ds(s * BLK, BLK), :],
                    dst_ref=out_ref.at[:, pl.ds(s * BLK, BLK)],
                    send_sem=send_sems.at[s],
                    recv_sem=recv_sems.at[s],
                    device_id=(s,),
                    device_id_type=pl.DeviceIdType.MESH,
                )
                recv.wait_recv()

        for t in range(N_DEV):
            @pl.when(t != me)
            def _():
                send = pltpu.make_async_remote_copy(
                    src_ref=x_ref.at[pl.ds(t * BLK, BLK), :],
                    dst_ref=out_ref.at[:, pl.ds(me * BLK, BLK)],
                    send_sem=send_sems.at[t],
                    recv_sem=recv_sems.at[me],
                    device_id=(t,),
                    device_id_type=pl.DeviceIdType.MESH,
                )
                send.wait_send()

    return pl.pallas_call(
        body,
        out_shape=jax.ShapeDtypeStruct((BLK, k_dim), x_shard.dtype),
        in_specs=[pl.BlockSpec(memory_space=pltpu.VMEM)],
        out_specs=pl.BlockSpec(memory_space=pltpu.VMEM),
        scratch_shapes=[
            pltpu.SemaphoreType.DMA((N_DEV,)),
            pltpu.SemaphoreType.DMA((N_DEV,)),
        ],
        compiler_params=pltpu.CompilerParams(collective_id=0),
    )(x_shard)


def _gemm_silu(x_rows, w_mat):
    m, k = x_rows.shape
    _, n = w_mat.shape

    def body(x_ref, w_ref, out_ref):
        y = jnp.dot(x_ref[:, :], w_ref[:, :], preferred_element_type=jnp.float32)
        out_ref[:, :] = y * jax.nn.sigmoid(y)

    grid = (n // N_BLK,)
    return pl.pallas_call(
        body,
        grid=grid,
        in_specs=[
            pl.BlockSpec((m, k), lambda j: (0, 0)),
            pl.BlockSpec((k, N_BLK), lambda j: (0, j)),
        ],
        out_specs=pl.BlockSpec((m, N_BLK), lambda j: (0, j)),
        out_shape=jax.ShapeDtypeStruct((m, n), jnp.float32),
    )(x_rows, w_mat)


def kernel(x, w_mat):
    x_rows = _a2a_kernel(x)
    return _gemm_silu(x_rows, w_mat)


# baseline (device time: 84492 ns/iter reference)
import jax
import jax.numpy as jnp
from jax import lax
from jax.experimental import pallas as pl
from jax.experimental.pallas import tpu as pltpu

N_DEV = 32
BLK = 128
N_COLS = 8192
N_BLK = 512


def _a2a_kernel(x_shard):
    k_dim, k_per = x_shard.shape

    def body(x_ref, out_ref, send_sems, recv_sems):
        me = lax.axis_index("i")

        for t in range(N_DEV):
            @pl.when(t != me)
            def _():
                rdma = pltpu.make_async_remote_copy(
                    src_ref=x_ref.at[pl.ds(t * BLK, BLK), :],
                    dst_ref=out_ref.at[:, pl.ds(me * BLK, BLK)],
                    send_sem=send_sems.at[t],
                    recv_sem=recv_sems.at[me],
                    device_id=(t,),
                    device_id_type=pl.DeviceIdType.MESH,
                )
                rdma.start()

        out_ref[:, pl.ds(me * BLK, BLK)] = x_ref[pl.ds(me * BLK, BLK), :]

        for s in range(N_DEV):
            @pl.when(s != me)
            def _():
                recv = pltpu.make_async_remote_copy(
                    src_ref=x_ref.at[pl.ds(s * BLK, BLK), :],
                    dst_ref=out_ref.at[:, pl.ds(s * BLK, BLK)],
                    send_sem=send_sems.at[s],
                    recv_sem=recv_sems.at[s],
                    device_id=(s,),
                    device_id_type=pl.DeviceIdType.MESH,
                )
                recv.wait_recv()

        for t in range(N_DEV):
            @pl.when(t != me)
            def _():
                send = pltpu.make_async_remote_copy(
                    src_ref=x_ref.at[pl.ds(t * BLK, BLK), :],
                    dst_ref=out_ref.at[:, pl.ds(me * BLK, BLK)],
                    send_sem=send_sems.at[t],
                    recv_sem=recv_sems.at[me],
                    device_id=(t,),
                    device_id_type=pl.DeviceIdType.MESH,
                )
                send.wait_send()

    return pl.pallas_call(
        body,
        out_shape=jax.ShapeDtypeStruct((BLK, k_dim), x_shard.dtype),
        in_specs=[pl.BlockSpec(memory_space=pltpu.VMEM)],
        out_specs=pl.BlockSpec(memory_space=pltpu.VMEM),
        scratch_shapes=[
            pltpu.SemaphoreType.DMA((N_DEV,)),
            pltpu.SemaphoreType.DMA((N_DEV,)),
        ],
    )(x_shard)


def _gemm_silu(x_rows, w_mat):
    m, k = x_rows.shape
    _, n = w_mat.shape

    def body(x_ref, w_ref, out_ref):
        y = jnp.dot(x_ref[:, :], w_ref[:, :], preferred_element_type=jnp.float32)
        out_ref[:, :] = y * jax.nn.sigmoid(y)

    grid = (n // N_BLK,)
    return pl.pallas_call(
        body,
        grid=grid,
        in_specs=[
            pl.BlockSpec((m, k), lambda j: (0, 0)),
            pl.BlockSpec((k, N_BLK), lambda j: (0, j)),
        ],
        out_specs=pl.BlockSpec((m, N_BLK), lambda j: (0, j)),
        out_shape=jax.ShapeDtypeStruct((m, n), jnp.float32),
    )(x_rows, w_mat)


def kernel(x, w_mat):
    x_rows = _a2a_kernel(x)
    return _gemm_silu(x_rows, w_mat)
